# baseline (speedup 1.0000x reference)
import functools

import jax
import jax.numpy as jnp
from jax import lax
from jax.experimental import pallas as pl
from jax.experimental.pallas import tpu as pltpu
from jax.experimental.pallas import tpu_sc as plsc

_LANES = 16
_HAND_LO = 33 * 3
_HAND_HI = 54 * 3


def _mask_body(n_pad, frames_per_w, idx_hbm, mask_hbm, idx_v, mask_v):
    num_cores = 2
    wid = lax.axis_index("s") * num_cores + lax.axis_index("c")
    lo = wid * frames_per_w

    pltpu.sync_copy(idx_hbm, idx_v)

    zeros = jnp.zeros((_LANES,), jnp.int32)

    @plsc.parallel_loop(0, frames_per_w, step=_LANES, unroll=8)
    def _zero(i):
        mask_v[pl.ds(i, _LANES)] = zeros

    ones = jnp.ones((_LANES,), jnp.int32)

    @plsc.parallel_loop(0, n_pad, step=_LANES, unroll=8)
    def _scan(i):
        v = idx_v[pl.ds(i, _LANES)]
        rel = v - lo
        inb = (rel >= 0) & (rel < frames_per_w)
        relc = jnp.clip(rel, 0, frames_per_w - 1)
        plsc.store_scatter(mask_v, [relc], ones, mask=inb)

    pltpu.sync_copy(mask_v, mask_hbm.at[pl.ds(lo, frames_per_w)])


def _build_mask(idx_padded, t):
    n_pad = idx_padded.shape[0]
    num_workers = 32
    frames_per_w = t // num_workers
    mesh = plsc.VectorSubcoreMesh(core_axis_name="c", subcore_axis_name="s")
    return pl.kernel(
        functools.partial(_mask_body, n_pad, frames_per_w),
        out_type=jax.ShapeDtypeStruct((t,), jnp.int32),
        mesh=mesh,
        scratch_types=[
            pltpu.VMEM((n_pad,), jnp.int32),
            pltpu.VMEM((frames_per_w,), jnp.int32),
        ],
        compiler_params=pltpu.CompilerParams(needs_layout_passes=False),
    )(idx_padded)


def _apply_body(mask_ref, x_ref, o_ref):
    shape = x_ref.shape
    m = mask_ref[0, 0, :]
    lmk = lax.broadcasted_iota(jnp.int32, shape, 1)
    hand = (lmk >= 33) & (lmk < 54)
    sel = hand & (m != 0)[None, None, :]
    o_ref[...] = jnp.where(sel, jnp.float32(jnp.nan), x_ref[...])


_BAND_LO = 32
_BAND_ROWS = 24


def _apply_body2(blk, mask_ref, x_any, o_any, band_in, band_out, sem_in, sem_out, sem_cp):
    i = pl.program_id(0)
    lanes = pl.ds(i * blk, blk)

    cps = []
    for rows in (pl.ds(0, _BAND_LO), pl.ds(_BAND_LO + _BAND_ROWS, 19)):
        cp = pltpu.make_async_copy(
            x_any.at[:, rows, lanes], o_any.at[:, rows, lanes], sem_cp
        )
        cp.start()
        cps.append(cp)

    band = pl.ds(_BAND_LO, _BAND_ROWS)
    cp_in = pltpu.make_async_copy(x_any.at[:, band, lanes], band_in, sem_in)
    cp_in.start()
    cp_in.wait()

    m = mask_ref[0, 0, :]
    lmk = lax.broadcasted_iota(jnp.int32, band_in.shape, 1) + _BAND_LO
    hand = (lmk >= 33) & (lmk < 54)
    sel = hand & (m != 0)[None, None, :]
    band_out[...] = jnp.where(sel, jnp.float32(jnp.nan), band_in[...])

    cp_out = pltpu.make_async_copy(band_out, o_any.at[:, band, lanes], sem_out)
    cp_out.start()
    cp_out.wait()
    for cp in cps:
        cp.wait()


def kernel(x, frame_indices):
    t, num_landmarks, coords = x.shape
    row = num_landmarks * coords
    n = frame_indices.shape[0]

    chunk = _LANES * 8
    n_pad = ((n + chunk - 1) // chunk) * chunk
    if n_pad != n:
        idx_padded = jnp.concatenate(
            [frame_indices, jnp.broadcast_to(frame_indices[:1], (n_pad - n,))]
        )
    else:
        idx_padded = frame_indices

    mask = _build_mask(idx_padded, t)

    blk = 8192
    grid = t // blk
    xt = jnp.transpose(x, (2, 1, 0))
    out = pl.pallas_call(
        functools.partial(_apply_body2, blk),
        grid=(grid,),
        in_specs=[
            pl.BlockSpec((1, 1, blk), lambda i: (i, 0, 0)),
            pl.BlockSpec(memory_space=pl.ANY),
        ],
        out_specs=pl.BlockSpec(memory_space=pl.ANY),
        out_shape=jax.ShapeDtypeStruct((coords, num_landmarks, t), jnp.float32),
        scratch_shapes=[
            pltpu.VMEM((coords, _BAND_ROWS, blk), jnp.float32),
            pltpu.VMEM((coords, _BAND_ROWS, blk), jnp.float32),
            pltpu.SemaphoreType.DMA,
            pltpu.SemaphoreType.DMA,
            pltpu.SemaphoreType.DMA,
        ],
    )(mask.reshape(grid, 1, blk), xt)
    return jnp.transpose(out, (2, 1, 0))


# speedup vs baseline: 26.2247x; 26.2247x over previous
import functools

import jax
import jax.numpy as jnp
from jax import lax
from jax.experimental import pallas as pl
from jax.experimental.pallas import tpu as pltpu
from jax.experimental.pallas import tpu_sc as plsc

_LANES = 16
_HAND_LO = 33 * 3
_HAND_HI = 54 * 3


def _mask_body(n_pad, frames_per_w, idx_hbm, mask_hbm, idx_v, mask_v):
    num_cores = 2
    wid = lax.axis_index("s") * num_cores + lax.axis_index("c")
    lo = wid * frames_per_w

    pltpu.sync_copy(idx_hbm, idx_v)

    zeros = jnp.zeros((_LANES,), jnp.int32)

    @plsc.parallel_loop(0, frames_per_w, step=_LANES, unroll=16)
    def _zero(i):
        mask_v[pl.ds(i, _LANES)] = zeros

    ones = jnp.ones((_LANES,), jnp.int32)

    @plsc.parallel_loop(0, n_pad, step=_LANES, unroll=16)
    def _scan(i):
        v = idx_v[pl.ds(i, _LANES)]
        rel = v - lo
        inb = (rel >= 0) & (rel < frames_per_w)
        relc = jnp.clip(rel, 0, frames_per_w - 1)
        plsc.store_scatter(mask_v, [relc], ones, mask=inb)

    pltpu.sync_copy(mask_v, mask_hbm.at[pl.ds(lo, frames_per_w)])


def _build_mask(idx_padded, t):
    n_pad = idx_padded.shape[0]
    num_workers = 32
    frames_per_w = t // num_workers
    mesh = plsc.VectorSubcoreMesh(core_axis_name="c", subcore_axis_name="s")
    return pl.kernel(
        functools.partial(_mask_body, n_pad, frames_per_w),
        out_type=jax.ShapeDtypeStruct((t,), jnp.int32),
        mesh=mesh,
        scratch_types=[
            pltpu.VMEM((n_pad,), jnp.int32),
            pltpu.VMEM((frames_per_w,), jnp.int32),
        ],
        compiler_params=pltpu.CompilerParams(needs_layout_passes=False),
    )(idx_padded)


def _apply_body(mask_ref, x_ref, o_ref):
    shape = x_ref.shape
    m = mask_ref[0, 0, :]
    lmk = lax.broadcasted_iota(jnp.int32, shape, 1)
    hand = (lmk >= 33) & (lmk < 54)
    sel = hand & (m != 0)[None, None, :]
    o_ref[...] = jnp.where(sel, jnp.float32(jnp.nan), x_ref[...])


def kernel(x, frame_indices):
    t, num_landmarks, coords = x.shape
    row = num_landmarks * coords
    n = frame_indices.shape[0]

    chunk = _LANES * 16
    n_pad = ((n + chunk - 1) // chunk) * chunk
    if n_pad != n:
        idx_padded = jnp.concatenate(
            [frame_indices, jnp.broadcast_to(frame_indices[:1], (n_pad - n,))]
        )
    else:
        idx_padded = frame_indices

    mask = _build_mask(idx_padded, t)

    blk = 8192
    grid = t // blk
    xt = jnp.transpose(x, (2, 1, 0))
    out = pl.pallas_call(
        _apply_body,
        grid=(grid,),
        in_specs=[
            pl.BlockSpec((1, 1, blk), lambda i: (i, 0, 0)),
            pl.BlockSpec((coords, num_landmarks, blk), lambda i: (0, 0, i)),
        ],
        out_specs=pl.BlockSpec((coords, num_landmarks, blk), lambda i: (0, 0, i)),
        out_shape=jax.ShapeDtypeStruct((coords, num_landmarks, t), jnp.float32),
    )(mask.reshape(grid, 1, blk), xt)
    return jnp.transpose(out, (2, 1, 0))


# speedup vs baseline: 26.2452x; 1.0008x over previous
import functools

import jax
import jax.numpy as jnp
from jax import lax
from jax.experimental import pallas as pl
from jax.experimental.pallas import tpu as pltpu
from jax.experimental.pallas import tpu_sc as plsc

_LANES = 16


def _mask_body(n_pad, frames_per_w, idx_hbm, mask_hbm, idx_v, mask_v):
    num_cores = 2
    wid = lax.axis_index("s") * num_cores + lax.axis_index("c")
    lo = wid * frames_per_w

    pltpu.sync_copy(idx_hbm, idx_v)

    zeros = jnp.zeros((_LANES,), jnp.int32)

    @plsc.parallel_loop(0, frames_per_w, step=_LANES, unroll=16)
    def _zero(i):
        mask_v[pl.ds(i, _LANES)] = zeros

    ones = jnp.ones((_LANES,), jnp.int32)

    @plsc.parallel_loop(0, n_pad, step=_LANES, unroll=16)
    def _scan(i):
        v = idx_v[pl.ds(i, _LANES)]
        rel = v - lo
        inb = (rel >= 0) & (rel < frames_per_w)
        relc = jnp.clip(rel, 0, frames_per_w - 1)
        plsc.store_scatter(mask_v, [relc], ones, mask=inb)

    pltpu.sync_copy(mask_v, mask_hbm.at[pl.ds(lo, frames_per_w)])


def _build_mask(idx_padded, t):
    n_pad = idx_padded.shape[0]
    num_workers = 32
    frames_per_w = t // num_workers
    mesh = plsc.VectorSubcoreMesh(core_axis_name="c", subcore_axis_name="s")
    return pl.kernel(
        functools.partial(_mask_body, n_pad, frames_per_w),
        out_type=jax.ShapeDtypeStruct((t,), jnp.int32),
        mesh=mesh,
        scratch_types=[
            pltpu.VMEM((n_pad,), jnp.int32),
            pltpu.VMEM((frames_per_w,), jnp.int32),
        ],
        compiler_params=pltpu.CompilerParams(needs_layout_passes=False),
    )(idx_padded)


def _apply_body(mask_ref, x_ref, o_ref):
    shape = x_ref.shape
    m = mask_ref[0, 0, :]
    lmk = lax.broadcasted_iota(jnp.int32, shape, 1)
    hand = (lmk >= 33) & (lmk < 54)
    sel = hand & (m != 0)[None, None, :]
    o_ref[...] = jnp.where(sel, jnp.float32(jnp.nan), x_ref[...])


def kernel(x, frame_indices):
    t, num_landmarks, coords = x.shape
    n = frame_indices.shape[0]

    chunk = _LANES * 16
    n_pad = ((n + chunk - 1) // chunk) * chunk
    if n_pad != n:
        idx_padded = jnp.concatenate(
            [frame_indices, jnp.broadcast_to(frame_indices[:1], (n_pad - n,))]
        )
    else:
        idx_padded = frame_indices

    mask = _build_mask(idx_padded, t)

    blk = 8192
    grid = t // blk
    xt = jnp.transpose(x, (2, 1, 0))
    out = pl.pallas_call(
        _apply_body,
        grid=(grid,),
        in_specs=[
            pl.BlockSpec((1, 1, blk), lambda i: (i, 0, 0)),
            pl.BlockSpec((coords, num_landmarks, blk), lambda i: (0, 0, i)),
        ],
        out_specs=pl.BlockSpec((coords, num_landmarks, blk), lambda i: (0, 0, i)),
        out_shape=jax.ShapeDtypeStruct((coords, num_landmarks, t), jnp.float32),
    )(mask.reshape(grid, 1, blk), xt)
    return jnp.transpose(out, (2, 1, 0))
